# baseline (speedup 1.0000x reference)
import functools

import jax
import jax.numpy as jnp
from jax import lax
from jax.experimental import pallas as pl
from jax.experimental.pallas import tpu as pltpu
from jax.experimental.pallas import tpu_sc as plsc

_D = 768
_LSEQ = 10
_NSEG = 2
_NV = _D // 16
_B = 16384
_N = _B * _LSEQ
_NW = 32
_TPW = _N // _NW
_SPW = _TPW // _LSEQ
_C = 80
_CSEQ = _C // _LSEQ
_NCHUNK = _TPW // _C
_NSTRIP = 4
_KPS = _NV // _NSTRIP
_INV_D = 1.0 / _D
_EPS = 1e-5


def _newton_rsqrt(v):
    i = lax.bitcast_convert_type(v, jnp.int32)
    i = jnp.full((16,), 0x5F3759DF, jnp.int32) - lax.shift_right_logical(i, 1)
    y = lax.bitcast_convert_type(i, jnp.float32)
    for _ in range(4):
        y = y * (jnp.float32(1.5) - jnp.float32(0.5) * v * y * y)
    return y


@functools.partial(
    pl.kernel,
    out_type=jax.ShapeDtypeStruct((_B, _LSEQ, _D), jnp.float32),
    mesh=plsc.VectorSubcoreMesh(core_axis_name="c", subcore_axis_name="s"),
    compiler_params=pltpu.CompilerParams(
        needs_layout_passes=False, use_tc_tiling_on_sc=False),
    scratch_types=[
        pltpu.VMEM((2, _C), jnp.int32),
        pltpu.VMEM((2, _C, _D), jnp.float32),
        pltpu.VMEM((_D,), jnp.float32),
        pltpu.VMEM((_D,), jnp.float32),
        pltpu.VMEM((2, 16, 16), jnp.float32),
        pltpu.VMEM((2, _C), jnp.float32),
        pltpu.SemaphoreType.DMA,
        pltpu.SemaphoreType.DMA,
        pltpu.SemaphoreType.DMA,
        pltpu.SemaphoreType.DMA,
    ],
)
def _emb_ln(comb, gidx, gamma, beta, out,
            idx_v, rows_v, g_v, b_v, sums_v, stat_v,
            gsem0, gsem1, osem0, osem1):
    gsems = (gsem0, gsem1)
    osems = (osem0, osem1)
    wid = lax.axis_index("c") * 16 + lax.axis_index("s")
    base0 = wid * _TPW
    seq0 = wid * _SPW
    pltpu.sync_copy(gamma, g_v)
    pltpu.sync_copy(beta, b_v)

    lanes = lax.iota(jnp.int32, 16)
    zeros16 = jnp.zeros((16,), jnp.int32)
    ones16 = jnp.full((16,), 1, jnp.int32)

    def _start_gather(g, b):
        pltpu.sync_copy(gidx.at[pl.ds(base0 + g * _C, _C)], idx_v.at[b])
        pltpu.async_copy(comb.at[idx_v.at[b]], rows_v.at[b], gsems[b])

    def _wait_gather(b):
        pltpu.make_async_copy(
            comb.at[pl.ds(0, _C)], rows_v.at[b], gsems[b]).wait()

    def _start_out(g, b):
        sbase = seq0 + g * _CSEQ
        for s in range(_CSEQ):
            pltpu.async_copy(
                rows_v.at[b].at[pl.ds(s * _LSEQ, _LSEQ)],
                out.at[sbase + s], osems[b])

    def _drain_out(b):
        for s in range(_CSEQ):
            pltpu.make_async_copy(
                rows_v.at[b].at[pl.ds(s * _LSEQ, _LSEQ)],
                out.at[seq0], osems[b]).wait()

    def _compute_chunk(g, b):
        def group_body(gg, carry2):
            jbase = gg * 16

            def tok_sum(t, carry3):
                j = jbase + t
                s1 = jnp.zeros((16,), jnp.float32)
                s2 = jnp.zeros((16,), jnp.float32)
                for k in range(_NV):
                    v = rows_v[b, j, pl.ds(k * 16, 16)]
                    s1 = s1 + v
                    s2 = s2 + v * v
                col = jnp.full((16,), t, jnp.int32)
                plsc.store_scatter(sums_v, [zeros16, lanes, col], s1)
                plsc.store_scatter(sums_v, [ones16, lanes, col], s2)
                return carry3

            lax.fori_loop(0, 16, tok_sum, 0)

            s1v = sums_v[0, 0, :]
            s2v = sums_v[1, 0, :]
            for i in range(1, 16):
                s1v = s1v + sums_v[0, i, :]
                s2v = s2v + sums_v[1, i, :]
            mean_v = s1v * jnp.float32(_INV_D)
            var_v = s2v * jnp.float32(_INV_D) - mean_v * mean_v
            rs_v = _newton_rsqrt(var_v + jnp.float32(_EPS))
            stat_v[0, pl.ds(jbase, 16)] = mean_v
            stat_v[1, pl.ds(jbase, 16)] = rs_v
            return carry2

        lax.fori_loop(0, _C // 16, group_body, 0)

        for strip in range(_NSTRIP):
            gs = [g_v[pl.ds((strip * _KPS + k) * 16, 16)] for k in range(_KPS)]
            bs = [b_v[pl.ds((strip * _KPS + k) * 16, 16)] for k in range(_KPS)]

            def tok_norm(t, carry3, gs=gs, bs=bs, strip=strip):
                tt = jnp.full((16,), t, jnp.int32)
                mb = plsc.load_gather(stat_v, [zeros16, tt])
                rb = plsc.load_gather(stat_v, [ones16, tt])
                for k in range(_KPS):
                    sl = pl.ds((strip * _KPS + k) * 16, 16)
                    rows_v[b, t, sl] = (
                        rows_v[b, t, sl] - mb) * rb * gs[k] + bs[k]
                return carry3

            lax.fori_loop(0, _C, tok_norm, 0)

    _start_gather(0, 0)

    def outer_body(go, carry):
        for bb in range(2):
            g = go * 2 + bb
            nxt = bb ^ 1

            @pl.when(g + 1 < _NCHUNK)
            def _():
                @pl.when(g >= 1)
                def _():
                    _drain_out(nxt)
                _start_gather(g + 1, nxt)

            _wait_gather(bb)
            _compute_chunk(g, bb)
            _start_out(g, bb)
        return carry

    lax.fori_loop(0, _NCHUNK // 2, outer_body, 0)
    _drain_out(0)
    _drain_out(1)


def kernel(x, seg, tok_emb, pos_emb, seg_emb, ln_gamma, ln_beta):
    comb = (
        pos_emb[:, None, None, :]
        + seg_emb[None, :, None, :]
        + tok_emb[None, None, :, :]
    ).reshape(-1, _D)
    l_ids = jnp.arange(_LSEQ, dtype=jnp.int32)[None, :]
    gidx = ((l_ids * _NSEG + seg) * tok_emb.shape[0] + x).reshape(-1)
    return _emb_ln(comb, gidx, ln_gamma, ln_beta)


# speedup vs baseline: 1.0457x; 1.0457x over previous
import functools

import jax
import jax.numpy as jnp
from jax import lax
from jax.experimental import pallas as pl
from jax.experimental.pallas import tpu as pltpu
from jax.experimental.pallas import tpu_sc as plsc

_D = 768
_LSEQ = 10
_NSEG = 2
_NV = _D // 16
_B = 16384
_N = _B * _LSEQ
_NW = 32
_TPW = _N // _NW
_C = 32
_NCHUNK = _TPW // _C
_NSTRIP = 4
_KPS = _NV // _NSTRIP
_INV_D = 1.0 / _D
_EPS = 1e-5


def _newton_rsqrt(v):
    i = lax.bitcast_convert_type(v, jnp.int32)
    i = jnp.full((16,), 0x5F3759DF, jnp.int32) - lax.shift_right_logical(i, 1)
    y = lax.bitcast_convert_type(i, jnp.float32)
    for _ in range(4):
        y = y * (jnp.float32(1.5) - jnp.float32(0.5) * v * y * y)
    return y


@functools.partial(
    pl.kernel,
    out_type=jax.ShapeDtypeStruct((_N, _D), jnp.float32),
    mesh=plsc.VectorSubcoreMesh(core_axis_name="c", subcore_axis_name="s"),
    compiler_params=pltpu.CompilerParams(needs_layout_passes=False),
    scratch_types=[
        pltpu.VMEM((_TPW,), jnp.int32),
        pltpu.VMEM((4, _C, _D), jnp.float32),
        pltpu.VMEM((_D,), jnp.float32),
        pltpu.VMEM((_D,), jnp.float32),
        pltpu.VMEM((2, 16, 16), jnp.float32),
        pltpu.VMEM((2, _C), jnp.float32),
        pltpu.SemaphoreType.DMA,
        pltpu.SemaphoreType.DMA,
        pltpu.SemaphoreType.DMA,
        pltpu.SemaphoreType.DMA,
        pltpu.SemaphoreType.DMA,
        pltpu.SemaphoreType.DMA,
        pltpu.SemaphoreType.DMA,
        pltpu.SemaphoreType.DMA,
    ],
)
def _emb_ln(comb, gidx, gamma, beta, out,
            idx_v, rows_v, g_v, b_v, sums_v, stat_v,
            gsem0, gsem1, gsem2, gsem3, osem0, osem1, osem2, osem3):
    gsems = (gsem0, gsem1, gsem2, gsem3)
    osems = (osem0, osem1, osem2, osem3)
    wid = lax.axis_index("c") * 16 + lax.axis_index("s")
    base0 = wid * _TPW
    pltpu.sync_copy(gidx.at[pl.ds(base0, _TPW)], idx_v)
    pltpu.sync_copy(gamma, g_v)
    pltpu.sync_copy(beta, b_v)

    lanes = lax.iota(jnp.int32, 16)
    zeros16 = jnp.zeros((16,), jnp.int32)
    ones16 = jnp.full((16,), 1, jnp.int32)

    def _start_gather(g, b):
        pltpu.async_copy(
            comb.at[idx_v.at[pl.ds(g * _C, _C)]], rows_v.at[b], gsems[b])

    def _wait_gather(b):
        pltpu.make_async_copy(
            comb.at[pl.ds(0, _C)], rows_v.at[b], gsems[b]).wait()

    def _drain_out(b):
        pltpu.make_async_copy(
            rows_v.at[b], out.at[pl.ds(base0, _C)], osems[b]).wait()

    def _compute_chunk(g, b):
        def group_body(gg, carry2):
            jbase = gg * 16

            def tok_sum(t, carry3):
                j = jbase + t
                s1 = jnp.zeros((16,), jnp.float32)
                s2 = jnp.zeros((16,), jnp.float32)
                for k in range(_NV):
                    v = rows_v[b, j, pl.ds(k * 16, 16)]
                    s1 = s1 + v
                    s2 = s2 + v * v
                col = jnp.full((16,), t, jnp.int32)
                plsc.store_scatter(sums_v, [zeros16, lanes, col], s1)
                plsc.store_scatter(sums_v, [ones16, lanes, col], s2)
                return carry3

            lax.fori_loop(0, 16, tok_sum, 0)

            s1v = sums_v[0, 0, :]
            s2v = sums_v[1, 0, :]
            for i in range(1, 16):
                s1v = s1v + sums_v[0, i, :]
                s2v = s2v + sums_v[1, i, :]
            mean_v = s1v * jnp.float32(_INV_D)
            var_v = s2v * jnp.float32(_INV_D) - mean_v * mean_v
            rs_v = _newton_rsqrt(var_v + jnp.float32(_EPS))
            stat_v[0, pl.ds(jbase, 16)] = mean_v
            stat_v[1, pl.ds(jbase, 16)] = rs_v
            return carry2

        lax.fori_loop(0, _C // 16, group_body, 0)

        for strip in range(_NSTRIP):
            gs = [g_v[pl.ds((strip * _KPS + k) * 16, 16)] for k in range(_KPS)]
            bs = [b_v[pl.ds((strip * _KPS + k) * 16, 16)] for k in range(_KPS)]

            def tok_norm(t, carry3, gs=gs, bs=bs, strip=strip):
                tt = jnp.full((16,), t, jnp.int32)
                mb = plsc.load_gather(stat_v, [zeros16, tt])
                rb = plsc.load_gather(stat_v, [ones16, tt])
                for k in range(_KPS):
                    sl = pl.ds((strip * _KPS + k) * 16, 16)
                    rows_v[b, t, sl] = (
                        rows_v[b, t, sl] - mb) * rb * gs[k] + bs[k]
                return carry3

            lax.fori_loop(0, _C, tok_norm, 0)

    _start_gather(0, 0)
    _start_gather(1, 1)

    def outer_body(go, carry):
        for bb in range(4):
            g = go * 4 + bb

            @pl.when((g >= 2) & (g + 2 < _NCHUNK))
            def _():
                _drain_out((bb + 2) % 4)

            @pl.when(g + 2 < _NCHUNK)
            def _():
                _start_gather(g + 2, (bb + 2) % 4)

            _wait_gather(bb)
            _compute_chunk(g, bb)
            pltpu.async_copy(
                rows_v.at[bb], out.at[pl.ds(base0 + g * _C, _C)], osems[bb])
        return carry

    lax.fori_loop(0, _NCHUNK // 4, outer_body, 0)
    for b in range(4):
        _drain_out(b)


def kernel(x, seg, tok_emb, pos_emb, seg_emb, ln_gamma, ln_beta):
    comb = (
        pos_emb[:, None, None, :]
        + seg_emb[None, :, None, :]
        + tok_emb[None, None, :, :]
    ).reshape(-1, _D)
    l_ids = jnp.arange(_LSEQ, dtype=jnp.int32)[None, :]
    gidx = ((l_ids * _NSEG + seg) * tok_emb.shape[0] + x).reshape(-1)
    out = _emb_ln(comb, gidx, ln_gamma, ln_beta)
    return out.reshape(x.shape[0], x.shape[1], _D)
